# baseline (speedup 1.0000x reference)
import functools
import math

import jax
import jax.numpy as jnp
from jax import lax
from jax.experimental import pallas as pl
from jax.experimental.pallas import tpu as pltpu
from jax.experimental.pallas import tpu_sc as plsc

N = 10000
E = 320000
D = 128
EPS = 1e-5

RB = 1000
GB = 2560

SUB = 16
LROWS = E // 128
RPT = LROWS // SUB
TAILR = LROWS - RPT * SUB
CH = 2
NCH = RPT // CH
CHD = 6
RT = N // SUB


def _gelu(x):
    return 0.5 * x * (1.0 + lax.erf(x * (1.0 / math.sqrt(2.0))))


def _ln(x, g, b):
    m = jnp.mean(x, axis=-1, keepdims=True)
    v = jnp.mean((x - m) ** 2, axis=-1, keepdims=True)
    return (x - m) * lax.rsqrt(v + EPS) * g + b


def _proj_body(x_ref, w_ref, b_ref, g_ref, beta_ref, o_ref):
    h = _gelu(jnp.dot(x_ref[...], w_ref[...],
                      preferred_element_type=jnp.float32) + b_ref[...])
    o_ref[...] = _ln(h, g_ref[...], beta_ref[...])


def _proj(x, p):
    n, din = x.shape
    return pl.pallas_call(
        _proj_body,
        grid=(n // RB,),
        in_specs=[
            pl.BlockSpec((RB, din), lambda i: (i, 0)),
            pl.BlockSpec((din, D), lambda i: (0, 0)),
            pl.BlockSpec((1, D), lambda i: (0, 0)),
            pl.BlockSpec((1, D), lambda i: (0, 0)),
            pl.BlockSpec((1, D), lambda i: (0, 0)),
        ],
        out_specs=pl.BlockSpec((RB, D), lambda i: (i, 0)),
        out_shape=jax.ShapeDtypeStruct((n, D), jnp.float32),
    )(x, p['W'], p['b'].reshape(1, D), p['g'].reshape(1, D),
      p['beta'].reshape(1, D))


def _mats_body(h_ref, w_ref, m_ref, t_ref):
    mm = jnp.dot(h_ref[...], w_ref[...], preferred_element_type=jnp.float32)
    m_ref[0] = mm[:, :64]
    m_ref[1] = mm[:, 64:D]
    t_ref[...] = mm[:, D:]


def _mats(h, wsrc, wdst):
    wcat = jnp.concatenate([wsrc, wdst], axis=1)
    return pl.pallas_call(
        _mats_body,
        grid=(N // RB,),
        in_specs=[
            pl.BlockSpec((RB, D), lambda i: (i, 0)),
            pl.BlockSpec((D, 2 * D), lambda i: (0, 0)),
        ],
        out_specs=[
            pl.BlockSpec((2, RB, 64), lambda i: (0, i, 0)),
            pl.BlockSpec((RB, D), lambda i: (i, 0)),
        ],
        out_shape=[
            jax.ShapeDtypeStruct((2, N, 64), jnp.float32),
            jax.ShapeDtypeStruct((N, D), jnp.float32),
        ],
    )(h, wcat)


def _gate_body(a_ref, w1_ref, b1_ref, w2_ref, b2_ref, o_ref):
    hh = _gelu(jnp.dot(a_ref[...], w1_ref[...],
                       preferred_element_type=jnp.float32) + b1_ref[...])
    s = jnp.sum(hh * w2_ref[...], axis=-1, keepdims=True) + b2_ref[0, 0]
    o_ref[...] = jax.nn.sigmoid(s)


def _gates(edge_attr, p):
    de = edge_attr.shape[1]
    return pl.pallas_call(
        _gate_body,
        grid=(E // GB,),
        in_specs=[
            pl.BlockSpec((GB, de), lambda i: (i, 0)),
            pl.BlockSpec((de, D), lambda i: (0, 0)),
            pl.BlockSpec((1, D), lambda i: (0, 0)),
            pl.BlockSpec((1, D), lambda i: (0, 0)),
            pl.BlockSpec((1, 1), lambda i: (0, 0)),
        ],
        out_specs=pl.BlockSpec((GB, 1), lambda i: (i, 0)),
        out_shape=jax.ShapeDtypeStruct((E, 1), jnp.float32),
    )(edge_attr, p['gW1'], p['gb1'].reshape(1, D), p['gW2'].reshape(1, D),
      p['gb2'].reshape(1, 1))


def _post_body(h_ref, a0_ref, a1_ref, t_ref, deg_ref, bd_ref, lg_ref, lb_ref,
               fg_ref, fb_ref, o_ref):
    deg = jnp.maximum(deg_ref[:, 0:1], 1.0)
    agg = jnp.concatenate([a0_ref[...], a1_ref[...]], axis=-1)
    x = agg / deg + t_ref[...] + bd_ref[...]
    conv = _gelu(_ln(x, lg_ref[...], lb_ref[...]))
    o_ref[...] = _ln(h_ref[...] + conv, fg_ref[...], fb_ref[...])


def _post(h_prev, agg0, agg1, t, deg, bdst, ln_g, ln_b, fin_g, fin_b):
    vec = pl.BlockSpec((1, D), lambda i: (0, 0))
    return pl.pallas_call(
        _post_body,
        grid=(N // RB,),
        in_specs=[
            pl.BlockSpec((RB, D), lambda i: (i, 0)),
            pl.BlockSpec((RB, 64), lambda i: (i, 0)),
            pl.BlockSpec((RB, 64), lambda i: (i, 0)),
            pl.BlockSpec((RB, D), lambda i: (i, 0)),
            pl.BlockSpec((RB, 16), lambda i: (i, 0)),
            vec, vec, vec, vec, vec,
        ],
        out_specs=pl.BlockSpec((RB, D), lambda i: (i, 0)),
        out_shape=jax.ShapeDtypeStruct((N, D), jnp.float32),
    )(h_prev, agg0, agg1, t, deg, bdst.reshape(1, D), ln_g.reshape(1, D),
      ln_b.reshape(1, D), fin_g.reshape(1, D), fin_b.reshape(1, D))


_MESH = plsc.VectorSubcoreMesh(core_axis_name="c", subcore_axis_name="s")
_SC_PARAMS = pltpu.CompilerParams(use_tc_tiling_on_sc=False,
                                  needs_layout_passes=False)

C256 = E // 256
CPT = C256 // SUB
CTAIL = C256 - CPT * SUB


def _make_sc_half(hc):
  mesh1 = plsc.VectorSubcoreMesh(core_axis_name="c", subcore_axis_name="s",
                                 num_cores=1)

  @functools.partial(
    pl.kernel,
    out_type=[
        jax.ShapeDtypeStruct((N, 64), jnp.float32),
        jax.ShapeDtypeStruct((N, 64), jnp.float32),
    ],
    mesh=mesh1,
    scratch_types=[
        pltpu.VMEM((2, 3, 256), jnp.int32),
        pltpu.VMEM((2, 3, 256), jnp.int32),
        pltpu.VMEM((256,), jnp.int32),
        pltpu.VMEM((256,), jnp.int32),
        pltpu.VMEM((256, 64), jnp.float32),
        pltpu.VMEM((256, 64), jnp.float32),
        pltpu.SemaphoreType.DMA((2,)),
        pltpu.SemaphoreType.DMA((2,)),
        pltpu.SemaphoreType.DMA((2,)),
        pltpu.SemaphoreType.DMA((2,)),
        pltpu.SemaphoreType.DMA((2,)),
        pltpu.SemaphoreType.DMA((2,)),
        pltpu.VMEM_SHARED((N, 64), jnp.float32),
        pltpu.VMEM_SHARED((N, 64), jnp.float32),
    ],
    compiler_params=_SC_PARAMS)
  def sc_half(m_ui, m_iu, iui, iiu, zrows, agg_ui, agg_iu,
              ia, ib, dscr0, dscr1, rows0, rows1,
              gsem0, gsem1, ssem0, ssem1, isemA, isemB, acc_ui, acc_iu):
    c = hc
    s = lax.axis_index("s")
    r0 = s * RT
    gsem0, gsem1 = gsem0.at[0], gsem1.at[0]
    ssem0, ssem1 = ssem0.at[0], ssem1.at[0]
    isemA, isemB = isemA.at[0], isemB.at[0]

    pltpu.sync_copy(zrows, acc_ui.at[pl.ds(r0, RT)])
    pltpu.sync_copy(zrows, acc_iu.at[pl.ds(r0, RT)])
    plsc.subcore_barrier()

    def flow(m_hbm, i3, acc, gated):
        base = s * CPT

        def fire(ibuf, j, rw, gsem):
            pltpu.async_copy(m_hbm.at[c].at[ibuf.at[j, 0]], rw, gsem)

        def drain_g(ibuf, j, rw, gsem):
            pltpu.make_async_copy(m_hbm.at[c].at[ibuf.at[j, 0]],
                                  rw, gsem).wait()

        def scale(ibuf, j, rw):
            @pl.loop(0, 16)
            def _(grp):
                g16 = plsc.bitcast(ibuf[j, 2, pl.ds(grp * 16, 16)],
                                   jnp.float32)
                for i in range(16):
                    w = g16[i]
                    row = grp * 16 + i
                    for jj in range(4):
                        sl = pl.ds(jj * 16, 16)
                        rw[row, sl] = rw[row, sl] * w

        def scat(ibuf, j, rw, dscr, ssem):
            for cc in range(16):
                sl = pl.ds(cc * 16, 16)
                dscr[sl] = ibuf[j, 1, sl]
            pltpu.async_copy(rw, acc.at[dscr], ssem, add=True)

        def drain_s(rw, dscr, ssem):
            pltpu.make_async_copy(rw, acc.at[dscr], ssem).wait()

        def proc(ibuf, j, rw, dscr, ssem):
            if gated:
                scale(ibuf, j, rw)
            scat(ibuf, j, rw, dscr, ssem)

        pltpu.sync_copy(i3.at[pl.ds(base, 2)], ia)
        pltpu.async_copy(i3.at[pl.ds(base + 2, 2)], ib, isemB)
        fire(ia, 0, rows0, gsem0)

        @pl.loop(0, 19)
        def _(k):
            p0 = base + 4 * k
            drain_g(ia, 0, rows0, gsem0)

            @pl.when(k > 0)
            def _():
                drain_s(rows1, dscr1, ssem1)
            fire(ia, 1, rows1, gsem1)
            proc(ia, 0, rows0, dscr0, ssem0)
            pltpu.make_async_copy(i3.at[pl.ds(0, 2)], ib, isemB).wait()
            drain_g(ia, 1, rows1, gsem1)
            drain_s(rows0, dscr0, ssem0)
            fire(ib, 0, rows0, gsem0)
            proc(ia, 1, rows1, dscr1, ssem1)
            pltpu.async_copy(i3.at[pl.ds(p0 + 4, 2)], ia, isemA)
            drain_g(ib, 0, rows0, gsem0)
            drain_s(rows1, dscr1, ssem1)
            fire(ib, 1, rows1, gsem1)
            proc(ib, 0, rows0, dscr0, ssem0)
            drain_g(ib, 1, rows1, gsem1)
            proc(ib, 1, rows1, dscr1, ssem1)
            pltpu.async_copy(i3.at[pl.ds(p0 + 6, 2)], ib, isemB)
            pltpu.make_async_copy(i3.at[pl.ds(0, 2)], ia, isemA).wait()
            drain_s(rows0, dscr0, ssem0)
            fire(ia, 0, rows0, gsem0)

        drain_g(ia, 0, rows0, gsem0)
        drain_s(rows1, dscr1, ssem1)
        fire(ia, 1, rows1, gsem1)
        proc(ia, 0, rows0, dscr0, ssem0)
        drain_g(ia, 1, rows1, gsem1)
        proc(ia, 1, rows1, dscr1, ssem1)
        pltpu.make_async_copy(i3.at[pl.ds(0, 2)], ib, isemB).wait()
        drain_s(rows0, dscr0, ssem0)
        drain_s(rows1, dscr1, ssem1)

        @pl.when(s == 0)
        def _():
            pltpu.sync_copy(i3.at[pl.ds(SUB * CPT, 2)], ia)
            fire(ia, 0, rows0, gsem0)
            drain_g(ia, 0, rows0, gsem0)
            proc(ia, 0, rows0, dscr0, ssem0)
            fire(ia, 1, rows1, gsem1)
            drain_g(ia, 1, rows1, gsem1)
            proc(ia, 1, rows1, dscr1, ssem1)
            drain_s(rows0, dscr0, ssem0)
            drain_s(rows1, dscr1, ssem1)

    flow(m_ui, iui, acc_ui, True)
    flow(m_iu, iiu, acc_iu, False)

    plsc.subcore_barrier()
    pltpu.sync_copy(acc_ui.at[pl.ds(r0, RT)], agg_ui.at[pl.ds(r0, RT)])
    pltpu.sync_copy(acc_iu.at[pl.ds(r0, RT)], agg_iu.at[pl.ds(r0, RT)])

  return sc_half


_sc_half0 = _make_sc_half(0)
_sc_half1 = _make_sc_half(1)


@functools.partial(
    pl.kernel,
    out_type=[
        jax.ShapeDtypeStruct((N, 16), jnp.float32),
        jax.ShapeDtypeStruct((N, 16), jnp.float32),
    ],
    mesh=_MESH,
    scratch_types=[
        pltpu.VMEM((CHD, 256), jnp.int32),
        pltpu.VMEM((256, 16), jnp.float32),
        pltpu.SemaphoreType.DMA((2,)),
        pltpu.VMEM_SHARED((N, 16), jnp.float32),
    ],
    compiler_params=_SC_PARAMS)
def _sc_deg(d_ui, d_iu, z16, deg_ui_o, deg_iu_o, dbuf, ones, ssem, dacc):
    c = lax.axis_index("c")
    s = lax.axis_index("s")
    r0 = s * RT
    ssem = ssem.at[c]

    pltpu.sync_copy(z16, dacc.at[pl.ds(r0, RT)])

    @pl.loop(0, 256)
    def _(r):
        ones[r, :] = jnp.ones((16,), jnp.float32)

    plsc.subcore_barrier()

    def dflow(d2):
        @pl.loop(0, CPT // CHD)
        def _(i):
            row0 = s * CPT + i * CHD
            pltpu.sync_copy(d2.at[pl.ds(row0, CHD)], dbuf)
            for j in range(CHD):
                pltpu.async_copy(ones, dacc.at[dbuf.at[j]], ssem, add=True)
            for j in range(CHD):
                pltpu.make_async_copy(ones, dacc.at[dbuf.at[j]], ssem).wait()

        @pl.when(s == 0)
        def _():
            pltpu.sync_copy(d2.at[pl.ds(SUB * CPT, CTAIL)],
                            dbuf.at[pl.ds(0, CTAIL)])
            for j in range(CTAIL):
                pltpu.async_copy(ones, dacc.at[dbuf.at[j]], ssem, add=True)
            for j in range(CTAIL):
                pltpu.make_async_copy(ones, dacc.at[dbuf.at[j]], ssem).wait()

    @pl.when(c == 0)
    def _():
        dflow(d_ui)

    @pl.when(c == 1)
    def _():
        dflow(d_iu)

    plsc.subcore_barrier()

    @pl.when(c == 0)
    def _():
        pltpu.sync_copy(dacc.at[pl.ds(r0, RT)], deg_ui_o.at[pl.ds(r0, RT)])

    @pl.when(c == 1)
    def _():
        pltpu.sync_copy(dacc.at[pl.ds(r0, RT)], deg_iu_o.at[pl.ds(r0, RT)])


def kernel(x_user, x_item, edge_index_ui, edge_attr_ui, edge_index_iu,
           params):
    s_ui = edge_index_ui[0].reshape(C256, 256)
    d_ui = edge_index_ui[1].reshape(C256, 256)
    s_iu = edge_index_iu[0].reshape(C256, 256)
    d_iu = edge_index_iu[1].reshape(C256, 256)
    zrows = jnp.zeros((RT, 64), jnp.float32)
    z16 = jnp.zeros((RT, 16), jnp.float32)

    h_u = _proj(x_user, params['proj']['user'])
    h_i = _proj(x_item, params['proj']['item'])

    iuis = []
    for lp in params['layers']:
        g2 = jax.lax.bitcast_convert_type(
            _gates(edge_attr_ui, lp['ui']).reshape(C256, 256), jnp.int32)
        iuis.append(jnp.stack([s_ui, d_ui, g2], axis=1))
    iiu = jnp.stack([s_iu, d_iu, d_iu], axis=1)

    deg_ui, deg_iu = _sc_deg(d_ui, d_iu, z16)

    for li, lp in enumerate(params['layers']):
        m_ui, t_u = _mats(h_u, lp['ui']['Wsrc'], lp['iu']['Wdst'])
        m_iu, t_i = _mats(h_i, lp['iu']['Wsrc'], lp['ui']['Wdst'])
        agg_ui0, agg_iu0 = _sc_half0(m_ui, m_iu, iuis[li], iiu, zrows)
        agg_ui1, agg_iu1 = _sc_half1(m_ui, m_iu, iuis[li], iiu, zrows)
        fin = params['final']
        h_i_new = _post(h_i, agg_ui0, agg_ui1, t_i, deg_ui, lp['ui']['bdst'],
                        lp['ui']['ln_g'], lp['ui']['ln_b'],
                        fin['item']['g'], fin['item']['beta'])
        h_u_new = _post(h_u, agg_iu0, agg_iu1, t_u, deg_iu, lp['iu']['bdst'],
                        lp['iu']['ln_g'], lp['iu']['ln_b'],
                        fin['user']['g'], fin['user']['beta'])
        h_u, h_i = h_u_new, h_i_new

    return h_u, h_i


# speedup vs baseline: 1.1121x; 1.1121x over previous
import functools
import math

import jax
import jax.numpy as jnp
from jax import lax
from jax.experimental import pallas as pl
from jax.experimental.pallas import tpu as pltpu
from jax.experimental.pallas import tpu_sc as plsc

N = 10000
E = 320000
D = 128
EPS = 1e-5

RB = 1000
GB = 2560

SUB = 16
LROWS = E // 128
RPT = LROWS // SUB
TAILR = LROWS - RPT * SUB
CH = 2
NCH = RPT // CH
CHD = 6
RT = N // SUB


def _gelu(x):
    return 0.5 * x * (1.0 + lax.erf(x * (1.0 / math.sqrt(2.0))))


def _ln(x, g, b):
    m = jnp.mean(x, axis=-1, keepdims=True)
    v = jnp.mean((x - m) ** 2, axis=-1, keepdims=True)
    return (x - m) * lax.rsqrt(v + EPS) * g + b


def _proj_body(x_ref, w_ref, b_ref, g_ref, beta_ref, o_ref):
    h = _gelu(jnp.dot(x_ref[...], w_ref[...],
                      preferred_element_type=jnp.float32) + b_ref[...])
    o_ref[...] = _ln(h, g_ref[...], beta_ref[...])


def _proj(x, p):
    n, din = x.shape
    return pl.pallas_call(
        _proj_body,
        grid=(n // RB,),
        in_specs=[
            pl.BlockSpec((RB, din), lambda i: (i, 0)),
            pl.BlockSpec((din, D), lambda i: (0, 0)),
            pl.BlockSpec((1, D), lambda i: (0, 0)),
            pl.BlockSpec((1, D), lambda i: (0, 0)),
            pl.BlockSpec((1, D), lambda i: (0, 0)),
        ],
        out_specs=pl.BlockSpec((RB, D), lambda i: (i, 0)),
        out_shape=jax.ShapeDtypeStruct((n, D), jnp.float32),
    )(x, p['W'], p['b'].reshape(1, D), p['g'].reshape(1, D),
      p['beta'].reshape(1, D))


def _mats_body(h_ref, w_ref, m_ref, t_ref):
    mm = jnp.dot(h_ref[...], w_ref[...], preferred_element_type=jnp.float32)
    m_ref[0] = mm[:, :64]
    m_ref[1] = mm[:, 64:D]
    t_ref[...] = mm[:, D:]


def _mats(h, wsrc, wdst):
    wcat = jnp.concatenate([wsrc, wdst], axis=1)
    return pl.pallas_call(
        _mats_body,
        grid=(N // RB,),
        in_specs=[
            pl.BlockSpec((RB, D), lambda i: (i, 0)),
            pl.BlockSpec((D, 2 * D), lambda i: (0, 0)),
        ],
        out_specs=[
            pl.BlockSpec((2, RB, 64), lambda i: (0, i, 0)),
            pl.BlockSpec((RB, D), lambda i: (i, 0)),
        ],
        out_shape=[
            jax.ShapeDtypeStruct((2, N, 64), jnp.float32),
            jax.ShapeDtypeStruct((N, D), jnp.float32),
        ],
    )(h, wcat)


def _gate_body(a_ref, w1_ref, b1_ref, w2_ref, b2_ref, o_ref):
    hh = _gelu(jnp.dot(a_ref[...], w1_ref[...],
                       preferred_element_type=jnp.float32) + b1_ref[...])
    s = jnp.sum(hh * w2_ref[...], axis=-1, keepdims=True) + b2_ref[0, 0]
    o_ref[...] = jax.nn.sigmoid(s)


def _gates(edge_attr, p):
    de = edge_attr.shape[1]
    return pl.pallas_call(
        _gate_body,
        grid=(E // GB,),
        in_specs=[
            pl.BlockSpec((GB, de), lambda i: (i, 0)),
            pl.BlockSpec((de, D), lambda i: (0, 0)),
            pl.BlockSpec((1, D), lambda i: (0, 0)),
            pl.BlockSpec((1, D), lambda i: (0, 0)),
            pl.BlockSpec((1, 1), lambda i: (0, 0)),
        ],
        out_specs=pl.BlockSpec((GB, 1), lambda i: (i, 0)),
        out_shape=jax.ShapeDtypeStruct((E, 1), jnp.float32),
    )(edge_attr, p['gW1'], p['gb1'].reshape(1, D), p['gW2'].reshape(1, D),
      p['gb2'].reshape(1, 1))


def _post_body(h_ref, agg_ref, t_ref, deg_ref, bd_ref, lg_ref, lb_ref,
               fg_ref, fb_ref, o_ref):
    deg = jnp.maximum(deg_ref[:, 0:1], 1.0)
    agg = jnp.concatenate([agg_ref[0], agg_ref[1]], axis=-1)
    x = agg / deg + t_ref[...] + bd_ref[...]
    conv = _gelu(_ln(x, lg_ref[...], lb_ref[...]))
    o_ref[...] = _ln(h_ref[...] + conv, fg_ref[...], fb_ref[...])


def _post(h_prev, agg, t, deg, bdst, ln_g, ln_b, fin_g, fin_b):
    vec = pl.BlockSpec((1, D), lambda i: (0, 0))
    return pl.pallas_call(
        _post_body,
        grid=(N // RB,),
        in_specs=[
            pl.BlockSpec((RB, D), lambda i: (i, 0)),
            pl.BlockSpec((2, RB, 64), lambda i: (0, i, 0)),
            pl.BlockSpec((RB, D), lambda i: (i, 0)),
            pl.BlockSpec((RB, 16), lambda i: (i, 0)),
            vec, vec, vec, vec, vec,
        ],
        out_specs=pl.BlockSpec((RB, D), lambda i: (i, 0)),
        out_shape=jax.ShapeDtypeStruct((N, D), jnp.float32),
    )(h_prev, agg, t, deg, bdst.reshape(1, D), ln_g.reshape(1, D),
      ln_b.reshape(1, D), fin_g.reshape(1, D), fin_b.reshape(1, D))


_MESH = plsc.VectorSubcoreMesh(core_axis_name="c", subcore_axis_name="s")
_SC_PARAMS = pltpu.CompilerParams(use_tc_tiling_on_sc=False,
                                  needs_layout_passes=False)

C256 = E // 256
CPT = C256 // SUB
CTAIL = C256 - CPT * SUB


@functools.partial(
    pl.kernel,
    out_type=[
        jax.ShapeDtypeStruct((2, N, 64), jnp.float32),
        jax.ShapeDtypeStruct((2, N, 64), jnp.float32),
    ],
    mesh=_MESH,
    scratch_types=[
        pltpu.VMEM((2, 3, 256), jnp.int32),
        pltpu.VMEM((2, 3, 256), jnp.int32),
        pltpu.VMEM((256,), jnp.int32),
        pltpu.VMEM((256,), jnp.int32),
        pltpu.VMEM((256, 64), jnp.float32),
        pltpu.VMEM((256, 64), jnp.float32),
        pltpu.SemaphoreType.DMA((2,)),
        pltpu.SemaphoreType.DMA((2,)),
        pltpu.SemaphoreType.DMA((2,)),
        pltpu.SemaphoreType.DMA((2,)),
        pltpu.SemaphoreType.DMA((2,)),
        pltpu.SemaphoreType.DMA((2,)),
        pltpu.VMEM_SHARED((N, 64), jnp.float32),
        pltpu.VMEM_SHARED((N, 64), jnp.float32),
    ],
    compiler_params=_SC_PARAMS)
def _sc_layer(m_ui, m_iu, iui, iiu, zrows, agg_ui, agg_iu,
              ia, ib, dscr0, dscr1, rows0, rows1,
              gsem0, gsem1, ssem0, ssem1, isemA, isemB, acc_ui, acc_iu):
    c = lax.axis_index("c")
    s = lax.axis_index("s")
    r0 = s * RT
    gsem0, gsem1 = gsem0.at[c], gsem1.at[c]
    ssem0, ssem1 = ssem0.at[c], ssem1.at[c]
    isemA, isemB = isemA.at[c], isemB.at[c]

    pltpu.sync_copy(zrows, acc_ui.at[pl.ds(r0, RT)])
    pltpu.sync_copy(zrows, acc_iu.at[pl.ds(r0, RT)])
    plsc.subcore_barrier()

    def flow(m_hbm, i3, acc, gated):
        base = s * CPT

        def fire(ibuf, j, rw, gsem):
            pltpu.async_copy(m_hbm.at[c].at[ibuf.at[j, 0]], rw, gsem)

        def drain_g(ibuf, j, rw, gsem):
            pltpu.make_async_copy(m_hbm.at[c].at[ibuf.at[j, 0]],
                                  rw, gsem).wait()

        def scale(ibuf, j, rw):
            @pl.loop(0, 16)
            def _(grp):
                g16 = plsc.bitcast(ibuf[j, 2, pl.ds(grp * 16, 16)],
                                   jnp.float32)
                for i in range(16):
                    w = g16[i]
                    row = grp * 16 + i
                    for jj in range(4):
                        sl = pl.ds(jj * 16, 16)
                        rw[row, sl] = rw[row, sl] * w

        def scat(ibuf, j, rw, dscr, ssem):
            for cc in range(16):
                sl = pl.ds(cc * 16, 16)
                dscr[sl] = ibuf[j, 1, sl]
            pltpu.async_copy(rw, acc.at[dscr], ssem, add=True)

        def drain_s(rw, dscr, ssem):
            pltpu.make_async_copy(rw, acc.at[dscr], ssem).wait()

        def proc(ibuf, j, rw, dscr, ssem):
            if gated:
                scale(ibuf, j, rw)
            scat(ibuf, j, rw, dscr, ssem)

        @pl.loop(0, CPT // 2)
        def _(k):
            p0 = base + 2 * k
            pltpu.sync_copy(i3.at[pl.ds(p0, 2)], ia)
            for j in range(2):
                pltpu.sync_copy(m_hbm.at[c].at[ia.at[j, 0]], rows0)
                if gated:
                    scale(ia, j, rows0)
                pltpu.sync_copy(rows0, acc.at[ia.at[j, 1]], add=True)

        @pl.when(s == 0)
        def _():
            pltpu.sync_copy(i3.at[pl.ds(SUB * CPT, 2)], ia)
            for j in range(2):
                pltpu.sync_copy(m_hbm.at[c].at[ia.at[j, 0]], rows0)
                if gated:
                    scale(ia, j, rows0)
                pltpu.sync_copy(rows0, acc.at[ia.at[j, 1]], add=True)

    flow(m_ui, iui, acc_ui, True)
    flow(m_iu, iiu, acc_iu, False)

    plsc.subcore_barrier()
    pltpu.sync_copy(acc_ui.at[pl.ds(r0, RT)], agg_ui.at[c].at[pl.ds(r0, RT)])
    pltpu.sync_copy(acc_iu.at[pl.ds(r0, RT)], agg_iu.at[c].at[pl.ds(r0, RT)])


@functools.partial(
    pl.kernel,
    out_type=[
        jax.ShapeDtypeStruct((N, 16), jnp.float32),
        jax.ShapeDtypeStruct((N, 16), jnp.float32),
    ],
    mesh=_MESH,
    scratch_types=[
        pltpu.VMEM((CHD, 256), jnp.int32),
        pltpu.VMEM((256, 16), jnp.float32),
        pltpu.SemaphoreType.DMA((2,)),
        pltpu.VMEM_SHARED((N, 16), jnp.float32),
    ],
    compiler_params=_SC_PARAMS)
def _sc_deg(d_ui, d_iu, z16, deg_ui_o, deg_iu_o, dbuf, ones, ssem, dacc):
    c = lax.axis_index("c")
    s = lax.axis_index("s")
    r0 = s * RT
    ssem = ssem.at[c]

    pltpu.sync_copy(z16, dacc.at[pl.ds(r0, RT)])

    @pl.loop(0, 256)
    def _(r):
        ones[r, :] = jnp.ones((16,), jnp.float32)

    plsc.subcore_barrier()

    def dflow(d2):
        @pl.loop(0, CPT // CHD)
        def _(i):
            row0 = s * CPT + i * CHD
            pltpu.sync_copy(d2.at[pl.ds(row0, CHD)], dbuf)
            for j in range(CHD):
                pltpu.async_copy(ones, dacc.at[dbuf.at[j]], ssem, add=True)
            for j in range(CHD):
                pltpu.make_async_copy(ones, dacc.at[dbuf.at[j]], ssem).wait()

        @pl.when(s == 0)
        def _():
            pltpu.sync_copy(d2.at[pl.ds(SUB * CPT, CTAIL)],
                            dbuf.at[pl.ds(0, CTAIL)])
            for j in range(CTAIL):
                pltpu.async_copy(ones, dacc.at[dbuf.at[j]], ssem, add=True)
            for j in range(CTAIL):
                pltpu.make_async_copy(ones, dacc.at[dbuf.at[j]], ssem).wait()

    @pl.when(c == 0)
    def _():
        dflow(d_ui)

    @pl.when(c == 1)
    def _():
        dflow(d_iu)

    plsc.subcore_barrier()

    @pl.when(c == 0)
    def _():
        pltpu.sync_copy(dacc.at[pl.ds(r0, RT)], deg_ui_o.at[pl.ds(r0, RT)])

    @pl.when(c == 1)
    def _():
        pltpu.sync_copy(dacc.at[pl.ds(r0, RT)], deg_iu_o.at[pl.ds(r0, RT)])


def kernel(x_user, x_item, edge_index_ui, edge_attr_ui, edge_index_iu,
           params):
    s_ui = edge_index_ui[0].reshape(C256, 256)
    d_ui = edge_index_ui[1].reshape(C256, 256)
    s_iu = edge_index_iu[0].reshape(C256, 256)
    d_iu = edge_index_iu[1].reshape(C256, 256)
    zrows = jnp.zeros((RT, 64), jnp.float32)
    z16 = jnp.zeros((RT, 16), jnp.float32)

    h_u = _proj(x_user, params['proj']['user'])
    h_i = _proj(x_item, params['proj']['item'])

    iuis = []
    for lp in params['layers']:
        g2 = jax.lax.bitcast_convert_type(
            _gates(edge_attr_ui, lp['ui']).reshape(C256, 256), jnp.int32)
        iuis.append(jnp.stack([s_ui, d_ui, g2], axis=1))
    iiu = jnp.stack([s_iu, d_iu, d_iu], axis=1)

    deg_ui, deg_iu = _sc_deg(d_ui, d_iu, z16)

    for li, lp in enumerate(params['layers']):
        m_ui, t_u = _mats(h_u, lp['ui']['Wsrc'], lp['iu']['Wdst'])
        m_iu, t_i = _mats(h_i, lp['iu']['Wsrc'], lp['ui']['Wdst'])
        agg_ui, agg_iu = _sc_layer(m_ui, m_iu, iuis[li], iiu, zrows)
        fin = params['final']
        h_i_new = _post(h_i, agg_ui, t_i, deg_ui, lp['ui']['bdst'],
                        lp['ui']['ln_g'], lp['ui']['ln_b'],
                        fin['item']['g'], fin['item']['beta'])
        h_u_new = _post(h_u, agg_iu, t_u, deg_iu, lp['iu']['bdst'],
                        lp['iu']['ln_g'], lp['iu']['ln_b'],
                        fin['user']['g'], fin['user']['beta'])
        h_u, h_i = h_u_new, h_i_new

    return h_u, h_i


# speedup vs baseline: 1.4636x; 1.3161x over previous
import functools
import math

import jax
import jax.numpy as jnp
from jax import lax
from jax.experimental import pallas as pl
from jax.experimental.pallas import tpu as pltpu
from jax.experimental.pallas import tpu_sc as plsc

N = 10000
E = 320000
D = 128
EPS = 1e-5

RB = 1000
GB = 2560

SUB = 16
LROWS = E // 128
RPT = LROWS // SUB
TAILR = LROWS - RPT * SUB
CH = 2
NCH = RPT // CH
CHD = 6
RT = N // SUB


def _gelu(x):
    return 0.5 * x * (1.0 + lax.erf(x * (1.0 / math.sqrt(2.0))))


def _ln(x, g, b):
    m = jnp.mean(x, axis=-1, keepdims=True)
    v = jnp.mean((x - m) ** 2, axis=-1, keepdims=True)
    return (x - m) * lax.rsqrt(v + EPS) * g + b


def _proj_body(x_ref, w_ref, b_ref, g_ref, beta_ref, o_ref):
    h = _gelu(jnp.dot(x_ref[...], w_ref[...],
                      preferred_element_type=jnp.float32) + b_ref[...])
    o_ref[...] = _ln(h, g_ref[...], beta_ref[...])


def _proj(x, p):
    n, din = x.shape
    return pl.pallas_call(
        _proj_body,
        grid=(n // RB,),
        in_specs=[
            pl.BlockSpec((RB, din), lambda i: (i, 0)),
            pl.BlockSpec((din, D), lambda i: (0, 0)),
            pl.BlockSpec((1, D), lambda i: (0, 0)),
            pl.BlockSpec((1, D), lambda i: (0, 0)),
            pl.BlockSpec((1, D), lambda i: (0, 0)),
        ],
        out_specs=pl.BlockSpec((RB, D), lambda i: (i, 0)),
        out_shape=jax.ShapeDtypeStruct((n, D), jnp.float32),
    )(x, p['W'], p['b'].reshape(1, D), p['g'].reshape(1, D),
      p['beta'].reshape(1, D))


def _mats_body(h_ref, w_ref, m_ref, t_ref):
    mm = jnp.dot(h_ref[...], w_ref[...], preferred_element_type=jnp.float32)
    m_ref[0] = mm[:, :64]
    m_ref[1] = mm[:, 64:D]
    t_ref[...] = mm[:, D:]


def _mats(h, wsrc, wdst):
    wcat = jnp.concatenate([wsrc, wdst], axis=1)
    return pl.pallas_call(
        _mats_body,
        grid=(N // RB,),
        in_specs=[
            pl.BlockSpec((RB, D), lambda i: (i, 0)),
            pl.BlockSpec((D, 2 * D), lambda i: (0, 0)),
        ],
        out_specs=[
            pl.BlockSpec((2, RB, 64), lambda i: (0, i, 0)),
            pl.BlockSpec((RB, D), lambda i: (i, 0)),
        ],
        out_shape=[
            jax.ShapeDtypeStruct((2, N, 64), jnp.float32),
            jax.ShapeDtypeStruct((N, D), jnp.float32),
        ],
    )(h, wcat)


def _gate_body(a_ref, w1_ref, b1_ref, w2_ref, b2_ref, o_ref):
    hh = _gelu(jnp.dot(a_ref[...], w1_ref[...],
                       preferred_element_type=jnp.float32) + b1_ref[...])
    s = jnp.sum(hh * w2_ref[...], axis=-1, keepdims=True) + b2_ref[0, 0]
    o_ref[...] = jax.nn.sigmoid(s)


def _gates(edge_attr, p):
    de = edge_attr.shape[1]
    return pl.pallas_call(
        _gate_body,
        grid=(E // GB,),
        in_specs=[
            pl.BlockSpec((GB, de), lambda i: (i, 0)),
            pl.BlockSpec((de, D), lambda i: (0, 0)),
            pl.BlockSpec((1, D), lambda i: (0, 0)),
            pl.BlockSpec((1, D), lambda i: (0, 0)),
            pl.BlockSpec((1, 1), lambda i: (0, 0)),
        ],
        out_specs=pl.BlockSpec((GB, 1), lambda i: (i, 0)),
        out_shape=jax.ShapeDtypeStruct((E, 1), jnp.float32),
    )(edge_attr, p['gW1'], p['gb1'].reshape(1, D), p['gW2'].reshape(1, D),
      p['gb2'].reshape(1, 1))


def _post_body(h_ref, agg_ref, t_ref, deg_ref, bd_ref, lg_ref, lb_ref,
               fg_ref, fb_ref, o_ref):
    deg = jnp.maximum(deg_ref[:, 0:1], 1.0)
    agg = jnp.concatenate([agg_ref[0], agg_ref[1]], axis=-1)
    x = agg / deg + t_ref[...] + bd_ref[...]
    conv = _gelu(_ln(x, lg_ref[...], lb_ref[...]))
    o_ref[...] = _ln(h_ref[...] + conv, fg_ref[...], fb_ref[...])


def _post(h_prev, agg, t, deg, bdst, ln_g, ln_b, fin_g, fin_b):
    vec = pl.BlockSpec((1, D), lambda i: (0, 0))
    return pl.pallas_call(
        _post_body,
        grid=(N // RB,),
        in_specs=[
            pl.BlockSpec((RB, D), lambda i: (i, 0)),
            pl.BlockSpec((2, RB, 64), lambda i: (0, i, 0)),
            pl.BlockSpec((RB, D), lambda i: (i, 0)),
            pl.BlockSpec((RB, 16), lambda i: (i, 0)),
            vec, vec, vec, vec, vec,
        ],
        out_specs=pl.BlockSpec((RB, D), lambda i: (i, 0)),
        out_shape=jax.ShapeDtypeStruct((N, D), jnp.float32),
    )(h_prev, agg, t, deg, bdst.reshape(1, D), ln_g.reshape(1, D),
      ln_b.reshape(1, D), fin_g.reshape(1, D), fin_b.reshape(1, D))


_MESH = plsc.VectorSubcoreMesh(core_axis_name="c", subcore_axis_name="s")
_SC_PARAMS = pltpu.CompilerParams(use_tc_tiling_on_sc=False,
                                  needs_layout_passes=False)

C256 = E // 256
CPT = C256 // SUB
CTAIL = C256 - CPT * SUB


@functools.partial(
    pl.kernel,
    out_type=[
        jax.ShapeDtypeStruct((2, N, 64), jnp.float32),
        jax.ShapeDtypeStruct((2, N, 64), jnp.float32),
    ],
    mesh=_MESH,
    scratch_types=[
        pltpu.VMEM((2, 3, 256), jnp.int32),
        pltpu.VMEM((2, 3, 256), jnp.int32),
        pltpu.VMEM((256,), jnp.int32),
        pltpu.VMEM((256,), jnp.int32),
        pltpu.VMEM((256, 64), jnp.float32),
        pltpu.VMEM((256, 64), jnp.float32),
        pltpu.SemaphoreType.DMA((2,)),
        pltpu.SemaphoreType.DMA((2,)),
        pltpu.SemaphoreType.DMA((2,)),
        pltpu.SemaphoreType.DMA((2,)),
        pltpu.SemaphoreType.DMA((2,)),
        pltpu.SemaphoreType.DMA((2,)),
        pltpu.VMEM_SHARED((N, 64), jnp.float32),
        pltpu.VMEM_SHARED((N, 64), jnp.float32),
    ],
    compiler_params=_SC_PARAMS)
def _sc_layer(m_ui, m_iu, iui, iiu, zrows, agg_ui, agg_iu,
              ia, ib, dscr0, dscr1, rows0, rows1,
              gsem0, gsem1, ssem0, ssem1, isemA, isemB, acc_ui, acc_iu):
    c = lax.axis_index("c")
    s = lax.axis_index("s")
    r0 = s * RT
    gsem0, gsem1 = gsem0.at[c], gsem1.at[c]
    ssem0, ssem1 = ssem0.at[c], ssem1.at[c]
    isemA, isemB = isemA.at[c], isemB.at[c]

    pltpu.sync_copy(zrows, acc_ui.at[pl.ds(r0, RT)])
    pltpu.sync_copy(zrows, acc_iu.at[pl.ds(r0, RT)])
    plsc.subcore_barrier()

    def flow(m_hbm, i3, acc, gated):
        base = s * CPT

        def fire(ibuf, j, rw, gsem):
            pltpu.async_copy(m_hbm.at[c].at[ibuf.at[j, 0]], rw, gsem)

        def drain_g(ibuf, j, rw, gsem):
            pltpu.make_async_copy(m_hbm.at[c].at[ibuf.at[j, 0]],
                                  rw, gsem).wait()

        def scale(ibuf, j, rw):
            @plsc.parallel_loop(0, 16, unroll=2)
            def _(grp):
                g16 = plsc.bitcast(ibuf[j, 2, pl.ds(grp * 16, 16)],
                                   jnp.float32)
                for i in range(16):
                    w = g16[i]
                    row = grp * 16 + i
                    for jj in range(4):
                        sl = pl.ds(jj * 16, 16)
                        rw[row, sl] = rw[row, sl] * w

        def scat(ibuf, j, rw, dscr, ssem):
            for cc in range(16):
                sl = pl.ds(cc * 16, 16)
                dscr[sl] = ibuf[j, 1, sl]
            pltpu.async_copy(rw, acc.at[dscr], ssem, add=True)

        def drain_s(rw, dscr, ssem):
            pltpu.make_async_copy(rw, acc.at[dscr], ssem).wait()

        def proc(ibuf, j, rw, dscr, ssem):
            if gated:
                scale(ibuf, j, rw)
            scat(ibuf, j, rw, dscr, ssem)

        pltpu.sync_copy(i3.at[pl.ds(base, 2)], ia)
        pltpu.async_copy(i3.at[pl.ds(base + 2, 2)], ib, isemB)
        fire(ia, 0, rows0, gsem0)

        @pl.loop(0, 19)
        def _(k):
            p0 = base + 4 * k
            drain_g(ia, 0, rows0, gsem0)

            @pl.when(k > 0)
            def _():
                drain_s(rows1, dscr1, ssem1)
            fire(ia, 1, rows1, gsem1)
            proc(ia, 0, rows0, dscr0, ssem0)
            pltpu.make_async_copy(i3.at[pl.ds(0, 2)], ib, isemB).wait()
            drain_g(ia, 1, rows1, gsem1)
            drain_s(rows0, dscr0, ssem0)
            fire(ib, 0, rows0, gsem0)
            proc(ia, 1, rows1, dscr1, ssem1)
            pltpu.async_copy(i3.at[pl.ds(p0 + 4, 2)], ia, isemA)
            drain_g(ib, 0, rows0, gsem0)
            drain_s(rows1, dscr1, ssem1)
            fire(ib, 1, rows1, gsem1)
            proc(ib, 0, rows0, dscr0, ssem0)
            drain_g(ib, 1, rows1, gsem1)
            proc(ib, 1, rows1, dscr1, ssem1)
            pltpu.async_copy(i3.at[pl.ds(p0 + 6, 2)], ib, isemB)
            pltpu.make_async_copy(i3.at[pl.ds(0, 2)], ia, isemA).wait()
            drain_s(rows0, dscr0, ssem0)
            fire(ia, 0, rows0, gsem0)

        drain_g(ia, 0, rows0, gsem0)
        drain_s(rows1, dscr1, ssem1)
        fire(ia, 1, rows1, gsem1)
        proc(ia, 0, rows0, dscr0, ssem0)
        drain_g(ia, 1, rows1, gsem1)
        proc(ia, 1, rows1, dscr1, ssem1)
        pltpu.make_async_copy(i3.at[pl.ds(0, 2)], ib, isemB).wait()
        drain_s(rows0, dscr0, ssem0)
        drain_s(rows1, dscr1, ssem1)

        @pl.when(s == 0)
        def _():
            pltpu.sync_copy(i3.at[pl.ds(SUB * CPT, 2)], ia)
            fire(ia, 0, rows0, gsem0)
            drain_g(ia, 0, rows0, gsem0)
            proc(ia, 0, rows0, dscr0, ssem0)
            fire(ia, 1, rows1, gsem1)
            drain_g(ia, 1, rows1, gsem1)
            proc(ia, 1, rows1, dscr1, ssem1)
            drain_s(rows0, dscr0, ssem0)
            drain_s(rows1, dscr1, ssem1)

    flow(m_ui, iui, acc_ui, True)
    flow(m_iu, iiu, acc_iu, False)

    plsc.subcore_barrier()
    pltpu.sync_copy(acc_ui.at[pl.ds(r0, RT)], agg_ui.at[c].at[pl.ds(r0, RT)])
    pltpu.sync_copy(acc_iu.at[pl.ds(r0, RT)], agg_iu.at[c].at[pl.ds(r0, RT)])


@functools.partial(
    pl.kernel,
    out_type=[
        jax.ShapeDtypeStruct((N, 16), jnp.float32),
        jax.ShapeDtypeStruct((N, 16), jnp.float32),
    ],
    mesh=_MESH,
    scratch_types=[
        pltpu.VMEM((CHD, 256), jnp.int32),
        pltpu.VMEM((256, 16), jnp.float32),
        pltpu.SemaphoreType.DMA((2,)),
        pltpu.VMEM_SHARED((N, 16), jnp.float32),
    ],
    compiler_params=_SC_PARAMS)
def _sc_deg(d_ui, d_iu, z16, deg_ui_o, deg_iu_o, dbuf, ones, ssem, dacc):
    c = lax.axis_index("c")
    s = lax.axis_index("s")
    r0 = s * RT
    ssem = ssem.at[c]

    pltpu.sync_copy(z16, dacc.at[pl.ds(r0, RT)])

    @pl.loop(0, 256)
    def _(r):
        ones[r, :] = jnp.ones((16,), jnp.float32)

    plsc.subcore_barrier()

    def dflow(d2):
        @pl.loop(0, CPT // CHD)
        def _(i):
            row0 = s * CPT + i * CHD
            pltpu.sync_copy(d2.at[pl.ds(row0, CHD)], dbuf)
            for j in range(CHD):
                pltpu.async_copy(ones, dacc.at[dbuf.at[j]], ssem, add=True)
            for j in range(CHD):
                pltpu.make_async_copy(ones, dacc.at[dbuf.at[j]], ssem).wait()

        @pl.when(s == 0)
        def _():
            pltpu.sync_copy(d2.at[pl.ds(SUB * CPT, CTAIL)],
                            dbuf.at[pl.ds(0, CTAIL)])
            for j in range(CTAIL):
                pltpu.async_copy(ones, dacc.at[dbuf.at[j]], ssem, add=True)
            for j in range(CTAIL):
                pltpu.make_async_copy(ones, dacc.at[dbuf.at[j]], ssem).wait()

    @pl.when(c == 0)
    def _():
        dflow(d_ui)

    @pl.when(c == 1)
    def _():
        dflow(d_iu)

    plsc.subcore_barrier()

    @pl.when(c == 0)
    def _():
        pltpu.sync_copy(dacc.at[pl.ds(r0, RT)], deg_ui_o.at[pl.ds(r0, RT)])

    @pl.when(c == 1)
    def _():
        pltpu.sync_copy(dacc.at[pl.ds(r0, RT)], deg_iu_o.at[pl.ds(r0, RT)])


def kernel(x_user, x_item, edge_index_ui, edge_attr_ui, edge_index_iu,
           params):
    s_ui = edge_index_ui[0].reshape(C256, 256)
    d_ui = edge_index_ui[1].reshape(C256, 256)
    s_iu = edge_index_iu[0].reshape(C256, 256)
    d_iu = edge_index_iu[1].reshape(C256, 256)
    zrows = jnp.zeros((RT, 64), jnp.float32)
    z16 = jnp.zeros((RT, 16), jnp.float32)

    h_u = _proj(x_user, params['proj']['user'])
    h_i = _proj(x_item, params['proj']['item'])

    iuis = []
    for lp in params['layers']:
        g2 = jax.lax.bitcast_convert_type(
            _gates(edge_attr_ui, lp['ui']).reshape(C256, 256), jnp.int32)
        iuis.append(jnp.stack([s_ui, d_ui, g2], axis=1))
    iiu = jnp.stack([s_iu, d_iu, d_iu], axis=1)

    deg_ui, deg_iu = _sc_deg(d_ui, d_iu, z16)

    for li, lp in enumerate(params['layers']):
        m_ui, t_u = _mats(h_u, lp['ui']['Wsrc'], lp['iu']['Wdst'])
        m_iu, t_i = _mats(h_i, lp['iu']['Wsrc'], lp['ui']['Wdst'])
        agg_ui, agg_iu = _sc_layer(m_ui, m_iu, iuis[li], iiu, zrows)
        fin = params['final']
        h_i_new = _post(h_i, agg_ui, t_i, deg_ui, lp['ui']['bdst'],
                        lp['ui']['ln_g'], lp['ui']['ln_b'],
                        fin['item']['g'], fin['item']['beta'])
        h_u_new = _post(h_u, agg_iu, t_u, deg_iu, lp['iu']['bdst'],
                        lp['iu']['ln_g'], lp['iu']['ln_b'],
                        fin['user']['g'], fin['user']['beta'])
        h_u, h_i = h_u_new, h_i_new

    return h_u, h_i
